# Initial kernel scaffold; baseline (speedup 1.0000x reference)
import jax
import jax.numpy as jnp
from jax.experimental import pallas as pl


def kernel(q, k, v, Wq, bq, Wk, bk, Wv, bv, q_indices, k_indices):
    raise NotImplementedError("write your pallas kernel here")


# speedup vs baseline: 9.3287x; 9.3287x over previous
import functools
import math

import jax
import jax.numpy as jnp
from jax import lax
from jax.experimental import pallas as pl
from jax.experimental.pallas import tpu as pltpu
from jax.experimental.pallas import tpu_sc as plsc

_N, _E, _D, _H = 10000, 320000, 128, 4
_DH = _D // _H
_INV_SQRT = 1.0 / math.sqrt(_DH)

_NS = 16
_EPW = _E // _NS
_B = 16
_NB = _EPW // _B
_H0 = 5120
_DEN0 = 5248
_TR = 6016
_IPT = _TR // _NS
_DUMP = 5200
_CH = 8


def _proj_body(q_ref, k_ref, v_ref, wq_ref, wk_ref, wv_ref,
               bq_ref, bk_ref, bv_ref, qo_ref, ko_ref, vo_ref):
    dn = (((1,), (1,)), ((), ()))
    qo_ref[...] = lax.dot_general(q_ref[...], wq_ref[...], dn,
                                  preferred_element_type=jnp.float32) + bq_ref[...]
    ko_ref[...] = lax.dot_general(k_ref[...], wk_ref[...], dn,
                                  preferred_element_type=jnp.float32) + bk_ref[...]
    vo_ref[...] = lax.dot_general(v_ref[...], wv_ref[...], dn,
                                  preferred_element_type=jnp.float32) + bv_ref[...]


def _project(q, k, v, Wq, bq, Wk, bk, Wv, bv):
    blk = 1000
    grid = _N // blk
    row_spec = pl.BlockSpec((blk, _D), lambda i: (i, 0))
    full_spec = pl.BlockSpec((_D, _D), lambda i: (0, 0))
    bias_spec = pl.BlockSpec((1, _D), lambda i: (0, 0))
    out_sds = jax.ShapeDtypeStruct((_N, _D), jnp.float32)
    return pl.pallas_call(
        _proj_body,
        grid=(grid,),
        in_specs=[row_spec, row_spec, row_spec,
                  full_spec, full_spec, full_spec,
                  bias_spec, bias_spec, bias_spec],
        out_specs=[row_spec, row_spec, row_spec],
        out_shape=[out_sds, out_sds, out_sds],
    )(q, k, v, Wq, Wk, Wv,
      bq.reshape(1, _D), bk.reshape(1, _D), bv.reshape(1, _D))


_sc_mesh = plsc.VectorSubcoreMesh(core_axis_name="c", subcore_axis_name="s")


def _lane_shuffle(v, idx):
    return lax.gather(
        v, idx[:, None],
        lax.GatherDimensionNumbers(offset_dims=(), collapsed_slice_dims=(0,),
                                   start_index_map=(0,)),
        (1,), mode=lax.GatherScatterMode.PROMISE_IN_BOUNDS)


_EDGE_KERNEL_KWARGS = dict(
    mesh=_sc_mesh,
    out_type=jax.ShapeDtypeStruct((2 * _TR, _D), jnp.float32),
    scratch_types=[
        pltpu.VMEM((_B,), jnp.int32),
        pltpu.VMEM((_B,), jnp.int32),
        pltpu.VMEM((_B,), jnp.int32),
        pltpu.VMEM((_B,), jnp.int32),
        pltpu.VMEM((_B, _D), jnp.float32),
        pltpu.VMEM((_B, _D), jnp.float32),
        pltpu.VMEM((_B, _D), jnp.float32),
        pltpu.VMEM((_B, _D), jnp.float32),
        pltpu.VMEM((_B, _D), jnp.float32),
        pltpu.VMEM_SHARED((_TR, _D), jnp.float32),
        pltpu.SemaphoreType.DMA,
        pltpu.SemaphoreType.DMA,
        pltpu.SemaphoreType.DMA,
    ],
)


def _edge_body(qn_hbm, kn_hbm, vn_hbm, qi_hbm, ki_hbm,
               tab_out,
               qi_v, ki_v, qiT_v, qiTD_v, Qr, Kr, Vr, Sr, denS,
               tab, sem_q, sem_k, sem_v):
    cid = lax.axis_index("c")
    sid = lax.axis_index("s")
    lo = cid * _H0
    span = jnp.where(cid == 0, _H0, _N - _H0)
    zeros16 = jnp.zeros((16,), jnp.float32)

    def _zero_bufs(r, carry):
        for cchunk in range(_D // 16):
            Sr[r, pl.ds(cchunk * 16, 16)] = zeros16
        return carry
    lax.fori_loop(0, _B, _zero_bufs, 0)

    r0 = sid * _IPT
    for chunk in range(_IPT // _CH):
        pltpu.sync_copy(Sr.at[pl.ds(0, _CH)],
                        tab.at[pl.ds(r0 + chunk * _CH, _CH)])
    plsc.subcore_barrier()

    def _batch(b, carry):
        base = sid * _EPW + b * _B
        pltpu.sync_copy(qi_hbm.at[pl.ds(base, _B)], qi_v)
        pltpu.sync_copy(ki_hbm.at[pl.ds(base, _B)], ki_v)
        cq = pltpu.async_copy(qn_hbm.at[qi_v], Qr, sem_q)
        ck = pltpu.async_copy(kn_hbm.at[ki_v], Kr, sem_k)
        cv = pltpu.async_copy(vn_hbm.at[ki_v], Vr, sem_v)
        cq.wait()
        ck.wait()
        cv.wait()

        for ch in range(_B // 16):
            qv = qi_v[pl.ds(ch * 16, 16)]
            inr = (qv >= lo) & (qv < lo + span)
            qt = jnp.where(inr, qv - lo, _DUMP)
            qiT_v[pl.ds(ch * 16, 16)] = qt
            qiTD_v[pl.ds(ch * 16, 16)] = _DEN0 + lax.shift_right_logical(qt, 3)

        iota = lax.iota(jnp.int32, 16)
        neg = jnp.where(iota < _H, 0.0, -1e30)
        masks = [jnp.where(iota == h, 1.0, 0.0) for h in range(_H)]

        def _edge(e, ecarry):
            u = []
            for h in range(_H):
                t = (Qr[e, pl.ds(h * _DH, 16)] * Kr[e, pl.ds(h * _DH, 16)]
                     + Qr[e, pl.ds(h * _DH + 16, 16)] * Kr[e, pl.ds(h * _DH + 16, 16)])
                for stride in (8, 4, 2, 1):
                    t = t + _lane_shuffle(t, iota ^ stride)
                u.append(t)
            sv = (u[0] * masks[0] + u[1] * masks[1]
                  + u[2] * masks[2] + u[3] * masks[3]) * _INV_SQRT + neg
            wv = jnp.exp(sv)
            for h in range(_H):
                wh = _lane_shuffle(wv, jnp.full((16,), h, jnp.int32))
                Sr[e, pl.ds(h * _DH, 16)] = wh * Vr[e, pl.ds(h * _DH, 16)]
                Sr[e, pl.ds(h * _DH + 16, 16)] = wh * Vr[e, pl.ds(h * _DH + 16, 16)]
            qt = qiT_v[pl.ds(0, 16)]
            slot = jnp.bitwise_and(_lane_shuffle(qt, jnp.broadcast_to(e, (16,))), 7)
            for c8 in range(8):
                msk = jnp.where(slot == c8, 1.0, 0.0)
                denS[e, pl.ds(c8 * 16, 16)] = wv * msk
            return ecarry
        lax.fori_loop(0, _B, _edge, 0)

        pltpu.sync_copy(Sr, tab.at[qiT_v], add=True)
        pltpu.sync_copy(denS, tab.at[qiTD_v], add=True)
        return carry
    lax.fori_loop(0, _NB, _batch, 0)

    plsc.subcore_barrier()
    for chunk in range(_IPT // _CH):
        rr = r0 + chunk * _CH
        pltpu.sync_copy(tab.at[pl.ds(rr, _CH)], Sr.at[pl.ds(0, _CH)])
        pltpu.sync_copy(Sr.at[pl.ds(0, _CH)],
                        tab_out.at[pl.ds(cid * _TR + rr, _CH)])


_edge_pass = pl.kernel(_edge_body, **_EDGE_KERNEL_KWARGS)


def _combine_body(num_ref, den_ref, out_ref):
    ns = num_ref[...]
    ds_ = den_ref[...]
    for h in range(_H):
        out_ref[:, h * _DH:(h + 1) * _DH] = (
            ns[:, h * _DH:(h + 1) * _DH] / ds_[:, h:h + 1])


def _combine(num, den):
    blk = 1000
    grid = _N // blk
    return pl.pallas_call(
        _combine_body,
        grid=(grid,),
        in_specs=[pl.BlockSpec((blk, _D), lambda i: (i, 0)),
                  pl.BlockSpec((blk, 16), lambda i: (i, 0))],
        out_specs=pl.BlockSpec((blk, _D), lambda i: (i, 0)),
        out_shape=jax.ShapeDtypeStruct((_N, _D), jnp.float32),
    )(num, den)


def kernel(q, k, v, Wq, bq, Wk, bk, Wv, bv, q_indices, k_indices):
    qn, kn, vn = _project(q, k, v, Wq, bq, Wk, bk, Wv, bv)
    tab = _edge_pass(qn, kn, vn, q_indices, k_indices)
    num = jnp.concatenate([tab[:_H0], tab[_TR:_TR + (_N - _H0)]], axis=0)
    den0 = tab[_DEN0:_DEN0 + _H0 // 8].reshape(_H0, 16)
    den1 = tab[_TR + _DEN0:_TR + _DEN0 + (_N - _H0) // 8].reshape(_N - _H0, 16)
    den = jnp.concatenate([den0, den1], axis=0)
    return _combine(num, den)


# speedup vs baseline: 12.1787x; 1.3055x over previous
import functools
import math

import jax
import jax.numpy as jnp
from jax import lax
from jax.experimental import pallas as pl
from jax.experimental.pallas import tpu as pltpu
from jax.experimental.pallas import tpu_sc as plsc

_N, _E, _D, _H = 10000, 320000, 128, 4
_DH = _D // _H
_INV_SQRT = 1.0 / math.sqrt(_DH)

_NS = 16
_EPW = _E // _NS
_B = 32
_NB = _EPW // _B
_H0 = 5120
_DEN0 = 5248
_TR = 6016
_IPT = _TR // _NS
_DUMP = 5200
_CH = 8


def _proj_body(q_ref, k_ref, v_ref, wq_ref, wk_ref, wv_ref,
               bq_ref, bk_ref, bv_ref, qo_ref, ko_ref, vo_ref):
    dn = (((1,), (1,)), ((), ()))
    qo_ref[...] = lax.dot_general(q_ref[...], wq_ref[...], dn,
                                  preferred_element_type=jnp.float32) + bq_ref[...]
    ko_ref[...] = lax.dot_general(k_ref[...], wk_ref[...], dn,
                                  preferred_element_type=jnp.float32) + bk_ref[...]
    vo_ref[...] = lax.dot_general(v_ref[...], wv_ref[...], dn,
                                  preferred_element_type=jnp.float32) + bv_ref[...]


def _project(q, k, v, Wq, bq, Wk, bk, Wv, bv):
    blk = 1000
    grid = _N // blk
    row_spec = pl.BlockSpec((blk, _D), lambda i: (i, 0))
    full_spec = pl.BlockSpec((_D, _D), lambda i: (0, 0))
    bias_spec = pl.BlockSpec((1, _D), lambda i: (0, 0))
    out_sds = jax.ShapeDtypeStruct((_N, _D), jnp.float32)
    return pl.pallas_call(
        _proj_body,
        grid=(grid,),
        in_specs=[row_spec, row_spec, row_spec,
                  full_spec, full_spec, full_spec,
                  bias_spec, bias_spec, bias_spec],
        out_specs=[row_spec, row_spec, row_spec],
        out_shape=[out_sds, out_sds, out_sds],
    )(q, k, v, Wq, Wk, Wv,
      bq.reshape(1, _D), bk.reshape(1, _D), bv.reshape(1, _D))


_sc_mesh = plsc.VectorSubcoreMesh(core_axis_name="c", subcore_axis_name="s")


def _lane_shuffle(v, idx):
    return lax.gather(
        v, idx[:, None],
        lax.GatherDimensionNumbers(offset_dims=(), collapsed_slice_dims=(0,),
                                   start_index_map=(0,)),
        (1,), mode=lax.GatherScatterMode.PROMISE_IN_BOUNDS)


_EDGE_KERNEL_KWARGS = dict(
    mesh=_sc_mesh,
    out_type=jax.ShapeDtypeStruct((2 * _TR, _D), jnp.float32),
    scratch_types=[
        pltpu.VMEM((_B,), jnp.int32),
        pltpu.VMEM((_B,), jnp.int32),
        pltpu.VMEM((_B,), jnp.int32),
        pltpu.VMEM((_B,), jnp.int32),
        pltpu.VMEM((_B, _D), jnp.float32),
        pltpu.VMEM((_B, _D), jnp.float32),
        pltpu.VMEM((_B, _D), jnp.float32),
        pltpu.VMEM((_B, _D), jnp.float32),
        pltpu.VMEM((_B, _D), jnp.float32),
        pltpu.VMEM_SHARED((_TR, _D), jnp.float32),
        pltpu.SemaphoreType.DMA,
        pltpu.SemaphoreType.DMA,
        pltpu.SemaphoreType.DMA,
    ],
)


def _edge_body(qn_hbm, kn_hbm, vn_hbm, qi_hbm, ki_hbm,
               tab_out,
               qi_v, ki_v, qiT_v, qiTD_v, Qr, Kr, Vr, Sr, denS,
               tab, sem_q, sem_k, sem_v):
    cid = lax.axis_index("c")
    sid = lax.axis_index("s")
    lo = cid * _H0
    span = jnp.where(cid == 0, _H0, _N - _H0)
    zeros16 = jnp.zeros((16,), jnp.float32)

    def _zero_bufs(r, carry):
        for cchunk in range(_D // 16):
            Sr[r, pl.ds(cchunk * 16, 16)] = zeros16
        return carry
    lax.fori_loop(0, _B, _zero_bufs, 0)

    r0 = sid * _IPT
    for chunk in range(_IPT // _CH):
        pltpu.sync_copy(Sr.at[pl.ds(0, _CH)],
                        tab.at[pl.ds(r0 + chunk * _CH, _CH)])
    plsc.subcore_barrier()

    def _batch(b, carry):
        base = sid * _EPW + b * _B
        pltpu.sync_copy(qi_hbm.at[pl.ds(base, _B)], qi_v)
        pltpu.sync_copy(ki_hbm.at[pl.ds(base, _B)], ki_v)
        cq = pltpu.async_copy(qn_hbm.at[qi_v], Qr, sem_q)
        ck = pltpu.async_copy(kn_hbm.at[ki_v], Kr, sem_k)
        cv = pltpu.async_copy(vn_hbm.at[ki_v], Vr, sem_v)
        cq.wait()
        ck.wait()
        cv.wait()

        for ch in range(_B // 16):
            qv = qi_v[pl.ds(ch * 16, 16)]
            inr = (qv >= lo) & (qv < lo + span)
            qt = jnp.where(inr, qv - lo, _DUMP)
            qiT_v[pl.ds(ch * 16, 16)] = qt
            qiTD_v[pl.ds(ch * 16, 16)] = _DEN0 + lax.shift_right_logical(qt, 3)

        iota = lax.iota(jnp.int32, 16)
        neg = jnp.where(iota < _H, 0.0, -1e30)
        masks = [jnp.where(iota == h, 1.0, 0.0) for h in range(_H)]

        def _edge(e, ecarry):
            u = []
            for h in range(_H):
                t = (Qr[e, pl.ds(h * _DH, 16)] * Kr[e, pl.ds(h * _DH, 16)]
                     + Qr[e, pl.ds(h * _DH + 16, 16)] * Kr[e, pl.ds(h * _DH + 16, 16)])
                for stride in (8, 4, 2, 1):
                    t = t + _lane_shuffle(t, iota ^ stride)
                u.append(t)
            sv = (u[0] * masks[0] + u[1] * masks[1]
                  + u[2] * masks[2] + u[3] * masks[3]) * _INV_SQRT + neg
            wv = jnp.exp(sv)
            for h in range(_H):
                wh = _lane_shuffle(wv, jnp.full((16,), h, jnp.int32))
                Sr[e, pl.ds(h * _DH, 16)] = wh * Vr[e, pl.ds(h * _DH, 16)]
                Sr[e, pl.ds(h * _DH + 16, 16)] = wh * Vr[e, pl.ds(h * _DH + 16, 16)]
            qt = qiT_v[pl.ds(jnp.bitwise_and(e, ~15), 16)]
            slot = jnp.bitwise_and(
                _lane_shuffle(qt, jnp.broadcast_to(jnp.bitwise_and(e, 15), (16,))), 7)
            for c8 in range(8):
                msk = jnp.where(slot == c8, 1.0, 0.0)
                denS[e, pl.ds(c8 * 16, 16)] = wv * msk
            return ecarry
        lax.fori_loop(0, _B, _edge, 0)

        pltpu.sync_copy(Sr, tab.at[qiT_v], add=True)
        pltpu.sync_copy(denS, tab.at[qiTD_v], add=True)
        return carry
    lax.fori_loop(0, _NB, _batch, 0)

    plsc.subcore_barrier()
    for chunk in range(_IPT // _CH):
        rr = r0 + chunk * _CH
        pltpu.sync_copy(tab.at[pl.ds(rr, _CH)], Sr.at[pl.ds(0, _CH)])
        pltpu.sync_copy(Sr.at[pl.ds(0, _CH)],
                        tab_out.at[pl.ds(cid * _TR + rr, _CH)])


_edge_pass = pl.kernel(_edge_body, **_EDGE_KERNEL_KWARGS)


def _combine_body(num_ref, den_ref, out_ref):
    ns = num_ref[...]
    ds_ = den_ref[...]
    for h in range(_H):
        out_ref[:, h * _DH:(h + 1) * _DH] = (
            ns[:, h * _DH:(h + 1) * _DH] / ds_[:, h:h + 1])


def _combine(num, den):
    blk = 1000
    grid = _N // blk
    return pl.pallas_call(
        _combine_body,
        grid=(grid,),
        in_specs=[pl.BlockSpec((blk, _D), lambda i: (i, 0)),
                  pl.BlockSpec((blk, 16), lambda i: (i, 0))],
        out_specs=pl.BlockSpec((blk, _D), lambda i: (i, 0)),
        out_shape=jax.ShapeDtypeStruct((_N, _D), jnp.float32),
    )(num, den)


def kernel(q, k, v, Wq, bq, Wk, bk, Wv, bv, q_indices, k_indices):
    qn, kn, vn = _project(q, k, v, Wq, bq, Wk, bk, Wv, bv)
    tab = _edge_pass(qn, kn, vn, q_indices, k_indices)
    num = jnp.concatenate([tab[:_H0], tab[_TR:_TR + (_N - _H0)]], axis=0)
    den0 = tab[_DEN0:_DEN0 + _H0 // 8].reshape(_H0, 16)
    den1 = tab[_TR + _DEN0:_TR + _DEN0 + (_N - _H0) // 8].reshape(_N - _H0, 16)
    den = jnp.concatenate([den0, den1], axis=0)
    return _combine(num, den)
